# Initial kernel scaffold; baseline (speedup 1.0000x reference)
import jax
import jax.numpy as jnp
from jax.experimental import pallas as pl


def kernel(agt, ctx, u, v, params):
    raise NotImplementedError("write your pallas kernel here")


# speedup vs baseline: 7.0470x; 7.0470x over previous
import functools

import jax
import jax.numpy as jnp
from jax import lax
from jax.experimental import pallas as pl
from jax.experimental.pallas import tpu as pltpu
from jax.experimental.pallas import tpu_sc as plsc

N = 10000
D = 128
E = 320000
DPAD = 10240
NC = 2
NS = 16
NWK = NC * NS
EPW = E // NWK
C = 80
NCH = EPW // C
SLAB = 624
ROWS_PER_TILE = 640


def _gn(x, g, b, eps=1e-5):
    m = jnp.mean(x, axis=1, keepdims=True)
    v = jnp.mean((x - m) ** 2, axis=1, keepdims=True)
    return (x - m) / jnp.sqrt(v + eps) * g + b


def _stage1_body(agt_ref, ctx_ref, wq_ref, wk_ref, cw_ref, cg_ref, cb_ref,
                 q_ref, k_ref, cp_ref):
    a = agt_ref[...]
    c = ctx_ref[...]
    dn = (((1,), (1,)), ((), ()))
    q_ref[...] = lax.dot_general(a, wq_ref[...], dn,
                                 preferred_element_type=jnp.float32)
    k_ref[...] = lax.dot_general(c, wk_ref[...], dn,
                                 preferred_element_type=jnp.float32)
    cp = lax.dot_general(c, cw_ref[...], dn,
                         preferred_element_type=jnp.float32)
    cp_ref[...] = _gn(cp, cg_ref[...], cb_ref[...])


def _stage1(agt, ctx, p):
    f32 = jnp.float32
    return pl.pallas_call(
        _stage1_body,
        out_shape=[jax.ShapeDtypeStruct((N, D), f32)] * 3,
    )(agt, ctx, p['wq'], p['wk'], p['ctx_w'],
      p['ctx_g'].reshape(1, D), p['ctx_b'].reshape(1, D))


def _stage2_body(q_hbm, k_hbm, cp_hbm, u_hbm, v_hbm, agg_hbm, den_hbm,
                 ubuf, vbuf, qrows, krows, crows, wvals, zbuf, tbuf,
                 agg_sh, den_sh, sem0, sem1, sem2):
    cid = lax.axis_index("c")
    sid = lax.axis_index("s")
    wid = sid * NC + cid
    zero16 = jnp.zeros((16,), jnp.float32)

    def _zrow(r, _):
        for j in range(D // 16):
            crows[r, pl.ds(j * 16, 16)] = zero16
        return 0
    lax.fori_loop(0, C, _zrow, 0)
    for j in range(DPAD // NS // 16):
        zbuf[pl.ds(j * 16, 16)] = zero16
    zb = pl.multiple_of(sid * SLAB, 8)
    for t in range(ROWS_PER_TILE // C):
        pltpu.sync_copy(crows, agg_sh.at[pl.ds(zb + t * C, C)])
    dzb = pl.multiple_of(sid * (DPAD // NS), 8)
    pltpu.sync_copy(zbuf, den_sh.at[pl.ds(dzb, DPAD // NS)])
    plsc.subcore_barrier()

    eb = pl.multiple_of(wid * EPW, 8)

    def _chunk(i, _):
        off = pl.multiple_of(eb + i * C, 8)
        pltpu.sync_copy(u_hbm.at[pl.ds(off, C)], ubuf)
        pltpu.sync_copy(v_hbm.at[pl.ds(off, C)], vbuf)
        cq = pltpu.async_copy(q_hbm.at[vbuf], qrows, sem0)
        ck = pltpu.async_copy(k_hbm.at[ubuf], krows, sem1)
        cc = pltpu.async_copy(cp_hbm.at[ubuf], crows, sem2)
        cq.wait()
        ck.wait()
        cc.wait()

        def _group(g, _):
            base = pl.multiple_of(g * 16, 8)
            lane = lax.iota(jnp.int32, 16)
            for e in range(16):
                r = base + e
                acc = qrows[r, pl.ds(0, 16)] * krows[r, pl.ds(0, 16)]
                for j in range(1, 8):
                    acc = acc + (qrows[r, pl.ds(j * 16, 16)]
                                 * krows[r, pl.ds(j * 16, 16)])
                tbuf[e, pl.ds(0, 16)] = acc
            dv = plsc.load_gather(tbuf, [lane, jnp.zeros((16,), jnp.int32)])
            for j in range(1, 16):
                dv = dv + plsc.load_gather(
                    tbuf, [lane, jnp.full((16,), j, jnp.int32)])
            lv = jnp.where(dv >= 0.0, dv, dv * 0.1)
            wv = jnp.exp(lv)
            wvals[pl.ds(base, 16)] = wv
            for e in range(16):
                r = base + e
                we = wv[e]
                for j in range(8):
                    crows[r, pl.ds(j * 16, 16)] = (
                        crows[r, pl.ds(j * 16, 16)] * we)
            return 0

        lax.fori_loop(0, C // 16, _group, 0)
        pltpu.sync_copy(crows, agg_sh.at[vbuf], add=True)
        pltpu.sync_copy(wvals, den_sh.at[vbuf], add=True)
        return 0

    lax.fori_loop(0, NCH, _chunk, 0)
    plsc.subcore_barrier()

    for t in range(ROWS_PER_TILE // C):
        pltpu.sync_copy(agg_sh.at[pl.ds(zb + t * C, C)], crows)
        pltpu.sync_copy(crows, agg_hbm.at[cid, pl.ds(zb + t * C, C)])
    pltpu.sync_copy(den_sh.at[pl.ds(dzb, DPAD // NS)], zbuf)
    dob = pl.multiple_of(cid * DPAD + dzb, 8)
    pltpu.sync_copy(zbuf, den_hbm.at[pl.ds(dob, DPAD // NS)])


def _stage2(q, k, cp, u, v):
    f32 = jnp.float32
    mesh = plsc.VectorSubcoreMesh(core_axis_name="c", subcore_axis_name="s")
    kern = pl.kernel(
        _stage2_body,
        out_type=[jax.ShapeDtypeStruct((NC, N, D), f32),
                  jax.ShapeDtypeStruct((NC * DPAD,), f32)],
        mesh=mesh,
        compiler_params=pltpu.CompilerParams(needs_layout_passes=False),
        scratch_types=[
            pltpu.VMEM((C,), jnp.int32),
            pltpu.VMEM((C,), jnp.int32),
            pltpu.VMEM((C, D), f32),
            pltpu.VMEM((C, D), f32),
            pltpu.VMEM((C, D), f32),
            pltpu.VMEM((C,), f32),
            pltpu.VMEM((DPAD // NS,), f32),
            pltpu.VMEM((16, 16), f32),
            pltpu.VMEM_SHARED((N, D), f32),
            pltpu.VMEM_SHARED((DPAD,), f32),
            pltpu.SemaphoreType.DMA,
            pltpu.SemaphoreType.DMA,
            pltpu.SemaphoreType.DMA,
        ],
    )
    return kern(q, k, cp, u, v)


def _stage2b_body(agg_hbm, den_hbm, out_hbm, a0, a1, d0, d1, sem0, sem1):
    cid = lax.axis_index("c")
    sid = lax.axis_index("s")
    wid = sid * NC + cid

    nchunks = (N // C - wid + NWK - 1) // NWK

    def _chunk(k, _):
        c = wid + k * NWK
        off = pl.multiple_of(c * C, 8)
        ca0 = pltpu.async_copy(agg_hbm.at[0, pl.ds(off, C)], a0, sem0)
        ca1 = pltpu.async_copy(agg_hbm.at[1, pl.ds(off, C)], a1, sem1)
        pltpu.sync_copy(den_hbm.at[pl.ds(off, C)], d0)
        pltpu.sync_copy(den_hbm.at[pl.ds(pl.multiple_of(DPAD + off, 8), C)],
                        d1)
        ca0.wait()
        ca1.wait()

        def _group(g, _):
            base = pl.multiple_of(g * 16, 8)
            dv = d0[pl.ds(base, 16)] + d1[pl.ds(base, 16)] + 1e-16
            rv = 1.0 / dv
            for e in range(16):
                r = base + e
                re = rv[e]
                for j in range(8):
                    a0[r, pl.ds(j * 16, 16)] = (
                        a0[r, pl.ds(j * 16, 16)]
                        + a1[r, pl.ds(j * 16, 16)]) * re
            return 0

        lax.fori_loop(0, C // 16, _group, 0)
        pltpu.sync_copy(a0, out_hbm.at[pl.ds(off, C)])
        return 0

    lax.fori_loop(0, nchunks, _chunk, 0)


def _stage2b(agg, den):
    f32 = jnp.float32
    mesh = plsc.VectorSubcoreMesh(core_axis_name="c", subcore_axis_name="s")
    kern = pl.kernel(
        _stage2b_body,
        out_type=jax.ShapeDtypeStruct((N, D), f32),
        mesh=mesh,
        compiler_params=pltpu.CompilerParams(needs_layout_passes=False),
        scratch_types=[
            pltpu.VMEM((C, D), f32),
            pltpu.VMEM((C, D), f32),
            pltpu.VMEM((C,), f32),
            pltpu.VMEM((C,), f32),
            pltpu.SemaphoreType.DMA,
            pltpu.SemaphoreType.DMA,
        ],
    )
    return kern(agg, den)


def _stage3_body(agt_ref, aggn_ref, aw_ref, ag_ref, ab_ref, ng_ref, nb_ref,
                 lw_ref, lg_ref, lb_ref, out_ref):
    res = agt_ref[...]
    aggn = aggn_ref[...]
    dn = (((1,), (1,)), ((), ()))
    w1 = aw_ref[:, :D]
    w2 = aw_ref[:, D:]
    a = (lax.dot_general(res, w1, dn, preferred_element_type=jnp.float32)
         + lax.dot_general(aggn, w2, dn, preferred_element_type=jnp.float32))
    a = _gn(a, ag_ref[...], ab_ref[...])
    a = _gn(a, ng_ref[...], nb_ref[...])
    a = jnp.maximum(a, 0.0)
    a = lax.dot_general(a, lw_ref[...], dn, preferred_element_type=jnp.float32)
    a = _gn(a, lg_ref[...], lb_ref[...])
    out_ref[...] = jnp.maximum(a + res, 0.0)


def _stage3(agt, agg, p):
    return pl.pallas_call(
        _stage3_body,
        out_shape=jax.ShapeDtypeStruct((N, D), jnp.float32),
    )(agt, agg, p['agt_w'],
      p['agt_g'].reshape(1, D), p['agt_b'].reshape(1, D),
      p['norm_g'].reshape(1, D), p['norm_b'].reshape(1, D),
      p['lin_w'],
      p['lin_g'].reshape(1, D), p['lin_b'].reshape(1, D))


def kernel(agt, ctx, u, v, params):
    u = u.astype(jnp.int32)
    v = v.astype(jnp.int32)
    for p in params:
        q, k, cp = _stage1(agt, ctx, p)
        agg, den = _stage2(q, k, cp, u, v)
        aggn = _stage2b(agg, den)
        agt = _stage3(agt, aggn, p)
    return agt


# speedup vs baseline: 9.9877x; 1.4173x over previous
import functools

import jax
import jax.numpy as jnp
from jax import lax
from jax.experimental import pallas as pl
from jax.experimental.pallas import tpu as pltpu
from jax.experimental.pallas import tpu_sc as plsc

N = 10000
D = 128
E = 320000
DPAD = 10240
NC = 2
NS = 16
NWK = NC * NS
EPW = E // NWK
C = 80
NCH = EPW // C
SLAB = 624
ROWS_PER_TILE = 640


def _gn(x, g, b, eps=1e-5):
    m = jnp.mean(x, axis=1, keepdims=True)
    v = jnp.mean((x - m) ** 2, axis=1, keepdims=True)
    return (x - m) / jnp.sqrt(v + eps) * g + b


def _stage1_body(agt_ref, ctx_ref, wq_ref, wk_ref, cw_ref, cg_ref, cb_ref,
                 q_ref, k_ref, cp_ref):
    a = agt_ref[...]
    c = ctx_ref[...]
    dn = (((1,), (1,)), ((), ()))
    q_ref[...] = lax.dot_general(a, wq_ref[...], dn,
                                 preferred_element_type=jnp.float32)
    k_ref[...] = lax.dot_general(c, wk_ref[...], dn,
                                 preferred_element_type=jnp.float32)
    cp = lax.dot_general(c, cw_ref[...], dn,
                         preferred_element_type=jnp.float32)
    cp_ref[...] = _gn(cp, cg_ref[...], cb_ref[...])


def _stage1(agt, ctx, p):
    f32 = jnp.float32
    return pl.pallas_call(
        _stage1_body,
        out_shape=[jax.ShapeDtypeStruct((N, D), f32)] * 3,
    )(agt, ctx, p['wq'], p['wk'], p['ctx_w'],
      p['ctx_g'].reshape(1, D), p['ctx_b'].reshape(1, D))


def _stage2_body(q_hbm, k_hbm, cp_hbm, u_hbm, v_hbm, agg_hbm, den_hbm,
                 ubuf0, vbuf0, ubuf1, vbuf1, wvals, zbuf, tbuf,
                 qrows, krows, crows0, crows1,
                 agg_sh, den_sh,
                 usem0, vsem0, usem1, vsem1,
                 qsem, ksem, csem0, csem1):
    cid = lax.axis_index("c")
    sid = lax.axis_index("s")
    wid = sid * NC + cid
    zero16 = jnp.zeros((16,), jnp.float32)
    cb = ((crows0, csem0), (crows1, csem1))
    ib = ((ubuf0, vbuf0, usem0, vsem0), (ubuf1, vbuf1, usem1, vsem1))

    def _zrow(r, _):
        for j in range(D // 16):
            crows0[r, pl.ds(j * 16, 16)] = zero16
        return 0
    lax.fori_loop(0, C, _zrow, 0)
    for j in range(DPAD // NS // 16):
        zbuf[pl.ds(j * 16, 16)] = zero16
    zb = pl.multiple_of(sid * SLAB, 8)
    for t in range(ROWS_PER_TILE // C):
        pltpu.sync_copy(crows0, agg_sh.at[pl.ds(zb + t * C, C)])
    dzb = pl.multiple_of(sid * (DPAD // NS), 8)
    pltpu.sync_copy(zbuf, den_sh.at[pl.ds(dzb, DPAD // NS)])
    plsc.subcore_barrier()

    eb = pl.multiple_of(wid * EPW, 8)

    def _issue_idx(c, p):
        ub, vb, us, vs = ib[p]
        off = pl.multiple_of(eb + c * C, 8)
        pltpu.async_copy(u_hbm.at[pl.ds(off, C)], ub, us)
        pltpu.async_copy(v_hbm.at[pl.ds(off, C)], vb, vs)

    def _wait_idx(p):
        ub, vb, us, vs = ib[p]
        pltpu.make_async_copy(u_hbm.at[pl.ds(0, C)], ub, us).wait()
        pltpu.make_async_copy(v_hbm.at[pl.ds(0, C)], vb, vs).wait()

    def _issue_qk(p):
        ub, vb = ib[p][:2]
        pltpu.async_copy(q_hbm.at[vb], qrows, qsem)
        pltpu.async_copy(k_hbm.at[ub], krows, ksem)

    def _issue_c(p, pc):
        ub = ib[p][0]
        cr, cs = cb[pc]
        pltpu.async_copy(cp_hbm.at[ub], cr, cs)

    def _wait_qk():
        pltpu.make_async_copy(q_hbm.at[pl.ds(0, C)], qrows, qsem).wait()
        pltpu.make_async_copy(k_hbm.at[pl.ds(0, C)], krows, ksem).wait()

    def _wait_c(pc):
        cr, cs = cb[pc]
        pltpu.make_async_copy(cp_hbm.at[pl.ds(0, C)], cr, cs).wait()

    def _dots():
        def _group(g, _):
            base = pl.multiple_of(g * 16, 8)
            lane = lax.iota(jnp.int32, 16)
            for e in range(16):
                r = base + e
                acc = qrows[r, pl.ds(0, 16)] * krows[r, pl.ds(0, 16)]
                for j in range(1, 8):
                    acc = acc + (qrows[r, pl.ds(j * 16, 16)]
                                 * krows[r, pl.ds(j * 16, 16)])
                tbuf[e, pl.ds(0, 16)] = acc
            dv = plsc.load_gather(tbuf, [lane, jnp.zeros((16,), jnp.int32)])
            for j in range(1, 16):
                dv = dv + plsc.load_gather(
                    tbuf, [lane, jnp.full((16,), j, jnp.int32)])
            lv = jnp.where(dv >= 0.0, dv, dv * 0.1)
            wvals[pl.ds(base, 16)] = jnp.exp(lv)
            return 0

        lax.fori_loop(0, C // 16, _group, 0)

    def _scale_scatter(p, pc):
        vb = ib[p][1]
        cr = cb[pc][0]

        def _group(g, _):
            base = pl.multiple_of(g * 16, 8)
            wv = wvals[pl.ds(base, 16)]
            for e in range(16):
                r = base + e
                we = wv[e]
                for j in range(8):
                    cr[r, pl.ds(j * 16, 16)] = cr[r, pl.ds(j * 16, 16)] * we
            return 0

        lax.fori_loop(0, C // 16, _group, 0)
        pltpu.sync_copy(cr, agg_sh.at[vb], add=True)
        pltpu.sync_copy(wvals, den_sh.at[vb], add=True)

    _issue_idx(0, 0)
    _wait_idx(0)
    _issue_qk(0)
    _issue_c(0, 0)
    _issue_idx(1, 1)

    def _step(c, p):
        pc = p
        _wait_qk()
        _dots()

        @pl.when(c + 1 < NCH)
        def _():
            _wait_idx(1 - p)
            _issue_qk(1 - p)
            _issue_c(1 - p, 1 - pc)
        _wait_c(pc)
        _scale_scatter(p, pc)

        @pl.when(c + 2 < NCH)
        def _():
            _issue_idx(c + 2, p)

    def _pair(i2, _):
        c0 = i2 * 2
        _step(c0, 0)

        @pl.when(c0 + 1 < NCH)
        def _():
            _step(c0 + 1, 1)
        return 0

    lax.fori_loop(0, (NCH + 1) // 2, _pair, 0)
    plsc.subcore_barrier()

    for t in range(ROWS_PER_TILE // C):
        pltpu.sync_copy(agg_sh.at[pl.ds(zb + t * C, C)], crows0)
        pltpu.sync_copy(crows0, agg_hbm.at[cid, pl.ds(zb + t * C, C)])
    pltpu.sync_copy(den_sh.at[pl.ds(dzb, DPAD // NS)], zbuf)
    dob = pl.multiple_of(cid * DPAD + dzb, 8)
    pltpu.sync_copy(zbuf, den_hbm.at[pl.ds(dob, DPAD // NS)])


def _stage2(q, k, cp, u, v):
    f32 = jnp.float32
    mesh = plsc.VectorSubcoreMesh(core_axis_name="c", subcore_axis_name="s")
    kern = pl.kernel(
        _stage2_body,
        out_type=[jax.ShapeDtypeStruct((NC, N, D), f32),
                  jax.ShapeDtypeStruct((NC * DPAD,), f32)],
        mesh=mesh,
        compiler_params=pltpu.CompilerParams(needs_layout_passes=False),
        scratch_types=[
            pltpu.VMEM((C,), jnp.int32),
            pltpu.VMEM((C,), jnp.int32),
            pltpu.VMEM((C,), jnp.int32),
            pltpu.VMEM((C,), jnp.int32),
            pltpu.VMEM((C,), f32),
            pltpu.VMEM((DPAD // NS,), f32),
            pltpu.VMEM((16, 16), f32),
            pltpu.VMEM((C, D), f32),
            pltpu.VMEM((C, D), f32),
            pltpu.VMEM((C, D), f32),
            pltpu.VMEM((C, D), f32),
            pltpu.VMEM_SHARED((N, D), f32),
            pltpu.VMEM_SHARED((DPAD,), f32),
        ] + [pltpu.SemaphoreType.DMA] * 8,
    )
    return kern(q, k, cp, u, v)


def _stage2b_body(agg_hbm, den_hbm, out_hbm, a0, a1, d0, d1, sem0, sem1):
    cid = lax.axis_index("c")
    sid = lax.axis_index("s")
    wid = sid * NC + cid

    nchunks = (N // C - wid + NWK - 1) // NWK

    def _chunk(k, _):
        c = wid + k * NWK
        off = pl.multiple_of(c * C, 8)
        ca0 = pltpu.async_copy(agg_hbm.at[0, pl.ds(off, C)], a0, sem0)
        ca1 = pltpu.async_copy(agg_hbm.at[1, pl.ds(off, C)], a1, sem1)
        pltpu.sync_copy(den_hbm.at[pl.ds(off, C)], d0)
        pltpu.sync_copy(den_hbm.at[pl.ds(pl.multiple_of(DPAD + off, 8), C)],
                        d1)
        ca0.wait()
        ca1.wait()

        def _group(g, _):
            base = pl.multiple_of(g * 16, 8)
            dv = d0[pl.ds(base, 16)] + d1[pl.ds(base, 16)] + 1e-16
            rv = 1.0 / dv
            for e in range(16):
                r = base + e
                re = rv[e]
                for j in range(8):
                    a0[r, pl.ds(j * 16, 16)] = (
                        a0[r, pl.ds(j * 16, 16)]
                        + a1[r, pl.ds(j * 16, 16)]) * re
            return 0

        lax.fori_loop(0, C // 16, _group, 0)
        pltpu.sync_copy(a0, out_hbm.at[pl.ds(off, C)])
        return 0

    lax.fori_loop(0, nchunks, _chunk, 0)


def _stage2b(agg, den):
    f32 = jnp.float32
    mesh = plsc.VectorSubcoreMesh(core_axis_name="c", subcore_axis_name="s")
    kern = pl.kernel(
        _stage2b_body,
        out_type=jax.ShapeDtypeStruct((N, D), f32),
        mesh=mesh,
        compiler_params=pltpu.CompilerParams(needs_layout_passes=False),
        scratch_types=[
            pltpu.VMEM((C, D), f32),
            pltpu.VMEM((C, D), f32),
            pltpu.VMEM((C,), f32),
            pltpu.VMEM((C,), f32),
            pltpu.SemaphoreType.DMA,
            pltpu.SemaphoreType.DMA,
        ],
    )
    return kern(agg, den)


def _stage3_body(agt_ref, aggn_ref, aw_ref, ag_ref, ab_ref, ng_ref, nb_ref,
                 lw_ref, lg_ref, lb_ref, out_ref):
    res = agt_ref[...]
    aggn = aggn_ref[...]
    dn = (((1,), (1,)), ((), ()))
    w1 = aw_ref[:, :D]
    w2 = aw_ref[:, D:]
    a = (lax.dot_general(res, w1, dn, preferred_element_type=jnp.float32)
         + lax.dot_general(aggn, w2, dn, preferred_element_type=jnp.float32))
    a = _gn(a, ag_ref[...], ab_ref[...])
    a = _gn(a, ng_ref[...], nb_ref[...])
    a = jnp.maximum(a, 0.0)
    a = lax.dot_general(a, lw_ref[...], dn, preferred_element_type=jnp.float32)
    a = _gn(a, lg_ref[...], lb_ref[...])
    out_ref[...] = jnp.maximum(a + res, 0.0)


def _stage3(agt, aggn, p):
    return pl.pallas_call(
        _stage3_body,
        out_shape=jax.ShapeDtypeStruct((N, D), jnp.float32),
    )(agt, aggn, p['agt_w'],
      p['agt_g'].reshape(1, D), p['agt_b'].reshape(1, D),
      p['norm_g'].reshape(1, D), p['norm_b'].reshape(1, D),
      p['lin_w'],
      p['lin_g'].reshape(1, D), p['lin_b'].reshape(1, D))


def kernel(agt, ctx, u, v, params):
    u = u.astype(jnp.int32)
    v = v.astype(jnp.int32)
    for p in params:
        q, k, cp = _stage1(agt, ctx, p)
        agg, den = _stage2(q, k, cp, u, v)
        aggn = _stage2b(agg, den)
        agt = _stage3(agt, aggn, p)
    return agt
